# baseline (speedup 1.0000x reference)
import jax
from jax.experimental import pallas as pl
from jax.experimental.pallas import tpu as pltpu


_ROWS = 320000
_BLOCK_ROWS = 26672


def _copy_kernel(x_ref, o_ref):
    o_ref[...] = x_ref[...]


def kernel(x_j):
    grid = (pl.cdiv(_ROWS, _BLOCK_ROWS),)
    return pl.pallas_call(
        _copy_kernel,
        out_shape=jax.ShapeDtypeStruct(x_j.shape, x_j.dtype),
        grid=grid,
        in_specs=[pl.BlockSpec((_BLOCK_ROWS, 128), lambda i: (i, 0))],
        out_specs=pl.BlockSpec((_BLOCK_ROWS, 128), lambda i: (i, 0)),
        compiler_params=pltpu.CompilerParams(vmem_limit_bytes=67108864),
    )(x_j)


# speedup vs baseline: 1.0092x; 1.0092x over previous
import jax
from jax.experimental import pallas as pl
from jax.experimental.pallas import tpu as pltpu


_ROWS = 320000
_CHUNKS = [1600, 3200, 6400, 12800] + [27200] * 10 + [12800, 6400, 3200, 1600]
assert sum(_CHUNKS) == _ROWS
_OFFS = [sum(_CHUNKS[:i]) for i in range(len(_CHUNKS))]
_N_CHUNKS = len(_CHUNKS)
_N_BUF = 4
_SLOT_ROWS = max(_CHUNKS)
_W = 2


def _copy_kernel(x_hbm, o_hbm, buf, in_sems, out_sems):
    def in_copy(i, s):
        return pltpu.make_async_copy(
            x_hbm.at[pl.ds(_OFFS[i], _CHUNKS[i])],
            buf.at[s, pl.ds(0, _CHUNKS[i])],
            in_sems.at[s],
        )

    def out_copy(i, s):
        return pltpu.make_async_copy(
            buf.at[s, pl.ds(0, _CHUNKS[i])],
            o_hbm.at[pl.ds(_OFFS[i], _CHUNKS[i])],
            out_sems.at[s],
        )

    for s in range(_N_BUF):
        in_copy(s, s).start()
    for i in range(_N_CHUNKS):
        s = i % _N_BUF
        in_copy(i, s).wait()
        out_copy(i, s).start()
        j = i - _W
        if j >= 0 and j + _N_BUF < _N_CHUNKS:
            out_copy(j, j % _N_BUF).wait()
            in_copy(j + _N_BUF, j % _N_BUF).start()
    for i in range(max(_N_CHUNKS - _N_BUF, _W), _N_CHUNKS):
        out_copy(i, i % _N_BUF).wait()


def kernel(x_j):
    return pl.pallas_call(
        _copy_kernel,
        out_shape=jax.ShapeDtypeStruct(x_j.shape, x_j.dtype),
        in_specs=[pl.BlockSpec(memory_space=pl.ANY)],
        out_specs=pl.BlockSpec(memory_space=pl.ANY),
        scratch_shapes=[
            pltpu.VMEM((_N_BUF, _SLOT_ROWS, 128), jax.numpy.float32),
            pltpu.SemaphoreType.DMA((_N_BUF,)),
            pltpu.SemaphoreType.DMA((_N_BUF,)),
        ],
        compiler_params=pltpu.CompilerParams(vmem_limit_bytes=67108864),
    )(x_j)


# speedup vs baseline: 1.0094x; 1.0002x over previous
import jax
from jax.experimental import pallas as pl
from jax.experimental.pallas import tpu as pltpu


_ROWS = 320000
_CHUNKS = [1600, 3200, 6400, 12400] + [24800] * 11 + [12400, 6400, 3200, 1600]
assert sum(_CHUNKS) == _ROWS
_OFFS = [sum(_CHUNKS[:i]) for i in range(len(_CHUNKS))]
_N_CHUNKS = len(_CHUNKS)
_N_BUF = 5
_SLOT_ROWS = max(_CHUNKS)
_W = 3


def _copy_kernel(x_hbm, o_hbm, buf, in_sems, out_sems):
    def in_copy(i, s):
        return pltpu.make_async_copy(
            x_hbm.at[pl.ds(_OFFS[i], _CHUNKS[i])],
            buf.at[s, pl.ds(0, _CHUNKS[i])],
            in_sems.at[s],
        )

    def out_copy(i, s):
        return pltpu.make_async_copy(
            buf.at[s, pl.ds(0, _CHUNKS[i])],
            o_hbm.at[pl.ds(_OFFS[i], _CHUNKS[i])],
            out_sems.at[s],
        )

    for s in range(_N_BUF):
        in_copy(s, s).start()
    for i in range(_N_CHUNKS):
        s = i % _N_BUF
        in_copy(i, s).wait()
        out_copy(i, s).start()
        j = i - _W
        if j >= 0 and j + _N_BUF < _N_CHUNKS:
            out_copy(j, j % _N_BUF).wait()
            in_copy(j + _N_BUF, j % _N_BUF).start()
    for i in range(max(_N_CHUNKS - _N_BUF, _W), _N_CHUNKS):
        out_copy(i, i % _N_BUF).wait()


def kernel(x_j):
    return pl.pallas_call(
        _copy_kernel,
        out_shape=jax.ShapeDtypeStruct(x_j.shape, x_j.dtype),
        in_specs=[pl.BlockSpec(memory_space=pl.ANY)],
        out_specs=pl.BlockSpec(memory_space=pl.ANY),
        scratch_shapes=[
            pltpu.VMEM((_N_BUF, _SLOT_ROWS, 128), jax.numpy.float32),
            pltpu.SemaphoreType.DMA((_N_BUF,)),
            pltpu.SemaphoreType.DMA((_N_BUF,)),
        ],
        compiler_params=pltpu.CompilerParams(vmem_limit_bytes=67108864),
    )(x_j)
